# Initial kernel scaffold; baseline (speedup 1.0000x reference)
import jax
import jax.numpy as jnp
from jax.experimental import pallas as pl


def kernel(ppl, visual_hint, answer, W_vh, b_vh, W_dh, b_dh, w_topo, W_g1, b_g1, W_g2, b_g2, W_g3, b_g3):
    raise NotImplementedError("write your pallas kernel here")


# speedup vs baseline: 1.8505x; 1.8505x over previous
import functools

import jax
import jax.numpy as jnp
from jax.experimental import pallas as pl
from jax.experimental.pallas import tpu as pltpu

EPS_TOPO = 0.7
BB = 8


def _fused_kernel(ppl_ref, vh_ref, ans_ref, w_top_ref, w_ans_ref, w_mid_ref,
                  b_all_ref, w_topo_ref, e_ref, wg1_ref, bg1_ref, wg2_ref,
                  bg2_ref, wg3_ref, bg3_ref, out_ref, *, n_nodes, k_heads):
    f32 = jnp.float32
    p = ppl_ref[...]

    pf = jnp.dot(p, w_top_ref[...], preferred_element_type=f32)
    pf += vh_ref[...] * w_mid_ref[...]
    ansp = jnp.dot(ans_ref[...], w_ans_ref[...], preferred_element_type=f32)
    pf += jnp.dot(e_ref[...], ansp, preferred_element_type=f32)
    pf = jnp.maximum(pf + b_all_ref[...], 0.0)

    gs = []
    for k in range(k_heads):
        xh = pf * w_topo_ref[k:k + 1, :]
        nrm = jnp.sqrt(jnp.sum(xh * xh, axis=1, keepdims=True))
        gs.append(xh / (nrm + 1e-8))
    g_all = jnp.concatenate(gs, axis=1)

    p1 = jnp.dot(p, wg1_ref[...], preferred_element_type=f32)

    n = n_nodes
    dn_t = (((1,), (1,)), ((), ()))
    ls, h1s = [], []
    for b in range(BB):
        gb = g_all[b * n:(b + 1) * n, :]
        a_b = jax.lax.dot_general(gb, gb, dn_t,
                                  preferred_element_type=f32) * (1.0 / k_heads)
        a_b = jnp.where(a_b > EPS_TOPO, a_b, 0.0)
        rows = jax.lax.broadcasted_iota(jnp.int32, (n, n), 0)
        cols = jax.lax.broadcasted_iota(jnp.int32, (n, n), 1)
        a_hat = a_b + (rows == cols).astype(f32)
        deg = jnp.sum(a_hat, axis=1, keepdims=True)
        deg_t = jnp.sum(a_hat, axis=0, keepdims=True)
        lb = a_hat * jax.lax.rsqrt(deg + 1e-8) * jax.lax.rsqrt(deg_t + 1e-8)
        ls.append(lb)
        z1 = jnp.dot(lb, p1[b * n:(b + 1) * n, :], preferred_element_type=f32)
        h1s.append(jnp.maximum(z1 + bg1_ref[...], 0.0))

    h1 = jnp.concatenate(h1s, axis=0)
    z2 = jnp.dot(h1, wg2_ref[...], preferred_element_type=f32)
    h2s = [jnp.maximum(
        jnp.dot(ls[b], z2[b * n:(b + 1) * n, :], preferred_element_type=f32)
        + bg2_ref[...], 0.0) for b in range(BB)]
    h2 = jnp.concatenate(h2s, axis=0)
    z3 = jnp.dot(h2, wg3_ref[...], preferred_element_type=f32)
    encs = [jnp.dot(ls[b], z3[b * n:(b + 1) * n, :], preferred_element_type=f32)
            + bg3_ref[...] for b in range(BB)]
    out_ref[...] = jnp.concatenate(encs, axis=0)


def kernel(ppl, visual_hint, answer, W_vh, b_vh, W_dh, b_dh, w_topo, W_g1,
           b_g1, W_g2, b_g2, W_g3, b_g3):
    B, N, D = ppl.shape
    H = W_dh.shape[1]
    VH = W_vh.shape[1]
    DA = answer.shape[1]
    K = w_topo.shape[0]
    assert B % BB == 0

    w_top = W_dh[:D]
    w_mid = (W_vh @ W_dh[D:D + VH]).reshape(1, H)
    w_ans = W_dh[D + VH:]
    b_all = (b_dh + b_vh @ W_dh[D:D + VH]).reshape(1, H)

    ppl2d = ppl.reshape(B * N, D)
    vh2d = visual_hint.reshape(B * N, 1)
    e_mat = jnp.repeat(jnp.eye(BB, dtype=jnp.float32), N, axis=0)

    R = BB * N
    grid = (B // BB,)
    const = lambda shape: pl.BlockSpec(shape, lambda i: (0, 0))
    out2d = pl.pallas_call(
        functools.partial(_fused_kernel, n_nodes=N, k_heads=K),
        grid=grid,
        in_specs=[
            pl.BlockSpec((R, D), lambda i: (i, 0)),
            pl.BlockSpec((R, 1), lambda i: (i, 0)),
            pl.BlockSpec((BB, DA), lambda i: (i, 0)),
            const((D, H)), const((DA, H)), const((1, H)), const((1, H)),
            const((K, H)), const((R, BB)),
            const((D, H)), const((1, H)),
            const((H, H)), const((1, H)),
            const((H, H)), const((1, H)),
        ],
        out_specs=pl.BlockSpec((R, H), lambda i: (i, 0)),
        out_shape=jax.ShapeDtypeStruct((B * N, H), jnp.float32),
        compiler_params=pltpu.CompilerParams(
            dimension_semantics=("arbitrary",)),
    )(ppl2d, vh2d, answer, w_top, w_ans, w_mid, b_all, w_topo, e_mat,
      W_g1, b_g1.reshape(1, H), W_g2, b_g2.reshape(1, H), W_g3,
      b_g3.reshape(1, H))
    return out2d.reshape(B, N, H)
